# baseline (speedup 1.0000x reference)
import functools

import jax
import jax.numpy as jnp
from jax import lax
from jax.experimental import pallas as pl
from jax.experimental.pallas import tpu as pltpu
from jax.experimental.pallas import tpu_sc as plsc

N = 10000
E = 320000
D = 128

NC = 2
NS = 16
NW = NC * NS
EPW = E // NW
B = 40
NBLK = EPW // B
RPS = N // NS
ZR = 125
ZRW = 25

_mesh = plsc.VectorSubcoreMesh(core_axis_name="c", subcore_axis_name="s")
_sc_params = pltpu.CompilerParams(use_tc_tiling_on_sc=False)


def _deg_body(dst_hbm, deg_hbm, didx_v, ones_v, zeros_v, hist_sh, sem):
    cid = lax.axis_index("c")
    sid = lax.axis_index("s")
    wid = sid * NC + cid

    @pl.loop(0, B, step=16)
    def _(i):
        ones_v[pl.ds(i, 16)] = jnp.ones((16,), jnp.float32)

    @pl.loop(0, 1000, step=16)
    def _(i):
        zeros_v[pl.ds(i, 16)] = jnp.zeros((16,), jnp.float32)

    @pl.when(sid < 10)
    def _():
        pltpu.sync_copy(zeros_v, hist_sh.at[pl.ds(sid * 1000, 1000)])

    plsc.subcore_barrier()

    pltpu.sync_copy(dst_hbm.at[wid], didx_v)

    @pl.loop(0, NBLK)
    def _(j):
        pltpu.async_copy(ones_v, hist_sh.at[didx_v.at[j]], sem, add=True)

    @pl.loop(0, NBLK)
    def _(j):
        pltpu.make_async_copy(ones_v, hist_sh.at[didx_v.at[j]], sem).wait()

    plsc.subcore_barrier()

    @pl.when(sid < 10)
    def _():
        row = cid * 10 + sid
        pltpu.sync_copy(hist_sh.at[pl.ds(sid * 1000, 1000)], deg_hbm.at[row])


@jax.jit
def _deg_parts(dst3):
    k = pl.kernel(
        _deg_body,
        out_type=jax.ShapeDtypeStruct((NC * 10, 1000), jnp.float32),
        mesh=_mesh,
        compiler_params=_sc_params,
        scratch_types=[
            pltpu.VMEM((NBLK, B), jnp.int32),
            pltpu.VMEM((B,), jnp.float32),
            pltpu.VMEM((1000,), jnp.float32),
            pltpu.VMEM_SHARED((N,), jnp.float32),
            pltpu.SemaphoreType.DMA,
        ],
    )
    return k(dst3).reshape(NC, N)


NBUF = 5
LEAD = 4
_T = NBLK // NBUF


def _edge_body(g_hbm, src_hbm, dst_hbm, acc_hbm,
               sidx_v, didx_v, zeros_v,
               r0, r1, r2, r3, r4,
               g0, g1, g2, g3, g4, s0, s1, s2, s3, s4, acc_sh):
    rows = (r0, r1, r2, r3, r4)
    gsems = (g0, g1, g2, g3, g4)
    ssems = (s0, s1, s2, s3, s4)
    cid = lax.axis_index("c")
    sid = lax.axis_index("s")
    wid = sid * NC + cid

    @pl.loop(0, ZRW)
    def _(r):
        @pl.loop(0, D, step=16)
        def _(c):
            zeros_v[r, pl.ds(c, 16)] = jnp.zeros((16,), jnp.float32)

    @pl.loop(0, RPS // ZRW)
    def _(kk):
        pltpu.sync_copy(zeros_v, acc_sh.at[pl.ds(sid * RPS + kk * ZRW, ZRW)])

    pltpu.sync_copy(src_hbm.at[wid], sidx_v)
    pltpu.sync_copy(dst_hbm.at[wid], didx_v)

    def _block(k, i, refill, wait_scatter):
        m = (i + LEAD) % NBUF
        if wait_scatter:
            pltpu.make_async_copy(rows[m],
                                  acc_sh.at[didx_v.at[k + LEAD - NBUF]],
                                  ssems[m]).wait()
        if refill:
            pltpu.async_copy(g_hbm.at[sidx_v.at[k + LEAD]], rows[m], gsems[m])
        pltpu.make_async_copy(g_hbm.at[sidx_v.at[k]], rows[i], gsems[i]).wait()
        pltpu.async_copy(rows[i], acc_sh.at[didx_v.at[k]], ssems[i], add=True)

    for i in range(LEAD):
        pltpu.async_copy(g_hbm.at[sidx_v.at[i]], rows[i], gsems[i])

    plsc.subcore_barrier()

    for k in range(NBUF):
        _block(k, k, refill=True, wait_scatter=(k + LEAD >= NBUF))

    @pl.loop(1, _T - 1)
    def _(t):
        for i in range(NBUF):
            _block(t * NBUF + i, i, refill=True, wait_scatter=True)

    for i in range(NBUF):
        k = NBLK - NBUF + i
        _block(k, i, refill=(k + LEAD < NBLK),
               wait_scatter=(k + LEAD < NBLK))

    for i in range(NBUF):
        pltpu.make_async_copy(rows[i], acc_sh.at[didx_v.at[NBLK - NBUF + i]],
                              ssems[i]).wait()

    plsc.subcore_barrier()

    @pl.loop(0, RPS // ZR)
    def _(kk):
        off = sid * RPS + kk * ZR
        pltpu.sync_copy(acc_sh.at[pl.ds(off, ZR)],
                        acc_hbm.at[pl.ds(cid * N + off, ZR)])


@jax.jit
def _edge_pass(g, src3, dst3):
    k = pl.kernel(
        _edge_body,
        out_type=jax.ShapeDtypeStruct((NC * N, D), jnp.float32),
        mesh=_mesh,
        compiler_params=_sc_params,
        scratch_types=[
            pltpu.VMEM((NBLK, B), jnp.int32),
            pltpu.VMEM((NBLK, B), jnp.int32),
            pltpu.VMEM((ZRW, D), jnp.float32),
        ] + [pltpu.VMEM((B, D), jnp.float32)] * NBUF
          + [pltpu.SemaphoreType.DMA] * (2 * NBUF)
          + [pltpu.VMEM_SHARED((N, D), jnp.float32)],
    )
    return k(g, src3, dst3).reshape(NC, N, D)


RB = 2000


def _dis(degp_ref):
    deg = degp_ref[:, 0] + degp_ref[:, 1] + 1.0
    return lax.rsqrt(deg)


def _mm_body(x_ref, w_ref, h_ref):
    h_ref[...] = jnp.dot(x_ref[...], w_ref[...],
                         preferred_element_type=jnp.float32)


def _scale_body(degp_ref, h_ref, g_ref):
    g_ref[...] = h_ref[...] * _dis(degp_ref)[:, None]


def _mid_body(degp_ref, a_ref, g_ref, b_ref, w_ref, g2_ref):
    dis = _dis(degp_ref)[:, None]
    out1 = (a_ref[0] + a_ref[1] + g_ref[...]) * dis + b_ref[...]
    h2 = jnp.dot(jnp.maximum(out1, 0.0), w_ref[...],
                 preferred_element_type=jnp.float32)
    g2_ref[...] = h2 * dis


def _post_body(degp_ref, a_ref, g_ref, b_ref, o_ref):
    dis = _dis(degp_ref)[:, None]
    o_ref[...] = (a_ref[0] + a_ref[1] + g_ref[...]) * dis + b_ref[...]


_spec_degp = pl.BlockSpec((RB, 2), lambda i: (i, 0))
_spec_rows = pl.BlockSpec((RB, D), lambda i: (i, 0))
_spec_acc = pl.BlockSpec((2, RB, D), lambda i: (0, i, 0))
_spec_w = pl.BlockSpec((D, D), lambda i: (0, 0))
_spec_b = pl.BlockSpec((1, D), lambda i: (0, 0))
_out_rows = jax.ShapeDtypeStruct((N, D), jnp.float32)


@jax.jit
def _tc_mm(x, w):
    return pl.pallas_call(
        _mm_body, grid=(N // RB,),
        in_specs=[_spec_rows, _spec_w],
        out_specs=_spec_rows, out_shape=_out_rows,
    )(x, w)


@jax.jit
def _tc_scale(degp, h):
    return pl.pallas_call(
        _scale_body, grid=(N // RB,),
        in_specs=[_spec_degp, _spec_rows],
        out_specs=_spec_rows, out_shape=_out_rows,
    )(degp, h)


@jax.jit
def _tc_mid(degp, acc, g, b, w):
    return pl.pallas_call(
        _mid_body, grid=(N // RB,),
        in_specs=[_spec_degp, _spec_acc, _spec_rows, _spec_b, _spec_w],
        out_specs=_spec_rows, out_shape=_out_rows,
    )(degp, acc, g, b, w)


@jax.jit
def _tc_post(degp, acc, g, b):
    return pl.pallas_call(
        _post_body, grid=(N // RB,),
        in_specs=[_spec_degp, _spec_acc, _spec_rows, _spec_b],
        out_specs=_spec_rows, out_shape=_out_rows,
    )(degp, acc, g, b)


def kernel(x, edge_index, W1, b1, W2, b2):
    src3 = edge_index[0].reshape(NW, NBLK, B)
    dst3 = edge_index[1].reshape(NW, NBLK, B)
    b1r = b1.reshape(1, D)
    b2r = b2.reshape(1, D)

    h1 = _tc_mm(x, W1)
    degp = _deg_parts(dst3).T
    g1 = _tc_scale(degp, h1)
    acc1 = _edge_pass(g1, src3, dst3)
    g2 = _tc_mid(degp, acc1, g1, b1r, W2)
    acc2 = _edge_pass(g2, src3, dst3)
    return _tc_post(degp, acc2, g2, b2r)


# speedup vs baseline: 1.0052x; 1.0052x over previous
import functools

import jax
import jax.numpy as jnp
from jax import lax
from jax.experimental import pallas as pl
from jax.experimental.pallas import tpu as pltpu
from jax.experimental.pallas import tpu_sc as plsc

N = 10000
E = 320000
D = 128

NC = 2
NS = 16
NW = NC * NS
EPW = E // NW
B = 40
NBLK = EPW // B
RPS = N // NS
ZR = 125
ZRW = 25

_mesh = plsc.VectorSubcoreMesh(core_axis_name="c", subcore_axis_name="s")
_sc_params = pltpu.CompilerParams(use_tc_tiling_on_sc=False)


def _deg_body(dst_hbm, deg_hbm, didx_v, ones_v, zeros_v, hist_sh, sem):
    cid = lax.axis_index("c")
    sid = lax.axis_index("s")
    wid = sid * NC + cid

    @pl.loop(0, B, step=16)
    def _(i):
        ones_v[pl.ds(i, 16)] = jnp.ones((16,), jnp.float32)

    @pl.loop(0, 1000, step=16)
    def _(i):
        zeros_v[pl.ds(i, 16)] = jnp.zeros((16,), jnp.float32)

    @pl.when(sid < 10)
    def _():
        pltpu.sync_copy(zeros_v, hist_sh.at[pl.ds(sid * 1000, 1000)])

    plsc.subcore_barrier()

    pltpu.sync_copy(dst_hbm.at[wid], didx_v)

    @pl.loop(0, NBLK)
    def _(j):
        pltpu.async_copy(ones_v, hist_sh.at[didx_v.at[j]], sem, add=True)

    @pl.loop(0, NBLK)
    def _(j):
        pltpu.make_async_copy(ones_v, hist_sh.at[didx_v.at[j]], sem).wait()

    plsc.subcore_barrier()

    @pl.when(sid < 10)
    def _():
        row = cid * 10 + sid
        pltpu.sync_copy(hist_sh.at[pl.ds(sid * 1000, 1000)], deg_hbm.at[row])


@jax.jit
def _deg_parts(dst3):
    k = pl.kernel(
        _deg_body,
        out_type=jax.ShapeDtypeStruct((NC * 10, 1000), jnp.float32),
        mesh=_mesh,
        compiler_params=_sc_params,
        scratch_types=[
            pltpu.VMEM((NBLK, B), jnp.int32),
            pltpu.VMEM((B,), jnp.float32),
            pltpu.VMEM((1000,), jnp.float32),
            pltpu.VMEM_SHARED((N,), jnp.float32),
            pltpu.SemaphoreType.DMA,
        ],
    )
    return k(dst3).reshape(NC, N)


NBUF = 5
LEAD = 4
_T = NBLK // NBUF


def _edge_body(g_hbm, src_hbm, dst_hbm, acc_hbm,
               sidx_v, didx_v, zeros_v,
               r0, r1, r2, r3, r4,
               g0, g1, g2, g3, g4, s0, s1, s2, s3, s4, acc_sh):
    rows = (r0, r1, r2, r3, r4)
    gsems = (g0, g1, g2, g3, g4)
    ssems = (s0, s1, s2, s3, s4)
    cid = lax.axis_index("c")
    sid = lax.axis_index("s")
    wid = sid * NC + cid

    @pl.loop(0, ZRW)
    def _(r):
        @pl.loop(0, D, step=16)
        def _(c):
            zeros_v[r, pl.ds(c, 16)] = jnp.zeros((16,), jnp.float32)

    @pl.loop(0, RPS // ZRW)
    def _(kk):
        pltpu.sync_copy(zeros_v, acc_sh.at[pl.ds(sid * RPS + kk * ZRW, ZRW)])

    pltpu.sync_copy(src_hbm.at[wid], sidx_v)
    pltpu.sync_copy(dst_hbm.at[wid], didx_v)

    def _block(k, i, refill, wait_scatter):
        m = (i + LEAD) % NBUF
        if wait_scatter:
            pltpu.make_async_copy(rows[m],
                                  acc_sh.at[didx_v.at[k + LEAD - NBUF]],
                                  ssems[m]).wait()
        if refill:
            pltpu.async_copy(g_hbm.at[sidx_v.at[k + LEAD]], rows[m], gsems[m])
        pltpu.make_async_copy(g_hbm.at[sidx_v.at[k]], rows[i], gsems[i]).wait()
        pltpu.async_copy(rows[i], acc_sh.at[didx_v.at[k]], ssems[i], add=True)

    for i in range(LEAD):
        pltpu.async_copy(g_hbm.at[sidx_v.at[i]], rows[i], gsems[i])

    plsc.subcore_barrier()

    for k in range(NBUF):
        _block(k, k, refill=True, wait_scatter=(k + LEAD >= NBUF))

    @pl.loop(1, _T - 1)
    def _(t):
        for i in range(NBUF):
            _block(t * NBUF + i, i, refill=True, wait_scatter=True)

    for i in range(NBUF):
        k = NBLK - NBUF + i
        _block(k, i, refill=(k + LEAD < NBLK),
               wait_scatter=(k + LEAD < NBLK))

    for i in range(NBUF):
        pltpu.make_async_copy(rows[i], acc_sh.at[didx_v.at[NBLK - NBUF + i]],
                              ssems[i]).wait()

    plsc.subcore_barrier()

    @pl.loop(0, RPS // ZR)
    def _(kk):
        off = sid * RPS + kk * ZR
        pltpu.sync_copy(acc_sh.at[pl.ds(off, ZR)],
                        acc_hbm.at[pl.ds(cid * N + off, ZR)])


@jax.jit
def _edge_pass(g, src3, dst3):
    k = pl.kernel(
        _edge_body,
        out_type=jax.ShapeDtypeStruct((NC * N, D), jnp.float32),
        mesh=_mesh,
        compiler_params=_sc_params,
        scratch_types=[
            pltpu.VMEM((NBLK, B), jnp.int32),
            pltpu.VMEM((NBLK, B), jnp.int32),
            pltpu.VMEM((ZRW, D), jnp.float32),
        ] + [pltpu.VMEM((B, D), jnp.float32)] * NBUF
          + [pltpu.SemaphoreType.DMA] * (2 * NBUF)
          + [pltpu.VMEM_SHARED((N, D), jnp.float32)],
    )
    return k(g, src3, dst3).reshape(NC, N, D)


RB = 2000


def _dis(degp_ref):
    deg = degp_ref[:, 0] + degp_ref[:, 1] + 1.0
    return lax.rsqrt(deg)


def _pre_body(degp_ref, x_ref, w_ref, g_ref):
    h = jnp.dot(x_ref[...], w_ref[...], preferred_element_type=jnp.float32)
    g_ref[...] = h * _dis(degp_ref)[:, None]


def _mid_body(degp_ref, a_ref, g_ref, b_ref, w_ref, g2_ref):
    dis = _dis(degp_ref)[:, None]
    out1 = (a_ref[0] + a_ref[1] + g_ref[...]) * dis + b_ref[...]
    h2 = jnp.dot(jnp.maximum(out1, 0.0), w_ref[...],
                 preferred_element_type=jnp.float32)
    g2_ref[...] = h2 * dis


def _post_body(degp_ref, a_ref, g_ref, b_ref, o_ref):
    dis = _dis(degp_ref)[:, None]
    o_ref[...] = (a_ref[0] + a_ref[1] + g_ref[...]) * dis + b_ref[...]


_spec_degp = pl.BlockSpec((RB, 2), lambda i: (i, 0))
_spec_rows = pl.BlockSpec((RB, D), lambda i: (i, 0))
_spec_acc = pl.BlockSpec((2, RB, D), lambda i: (0, i, 0))
_spec_w = pl.BlockSpec((D, D), lambda i: (0, 0))
_spec_b = pl.BlockSpec((1, D), lambda i: (0, 0))
_out_rows = jax.ShapeDtypeStruct((N, D), jnp.float32)


@jax.jit
def _tc_pre(degp, x, w):
    return pl.pallas_call(
        _pre_body, grid=(N // RB,),
        in_specs=[_spec_degp, _spec_rows, _spec_w],
        out_specs=_spec_rows, out_shape=_out_rows,
    )(degp, x, w)


@jax.jit
def _tc_mid(degp, acc, g, b, w):
    return pl.pallas_call(
        _mid_body, grid=(N // RB,),
        in_specs=[_spec_degp, _spec_acc, _spec_rows, _spec_b, _spec_w],
        out_specs=_spec_rows, out_shape=_out_rows,
    )(degp, acc, g, b, w)


@jax.jit
def _tc_post(degp, acc, g, b):
    return pl.pallas_call(
        _post_body, grid=(N // RB,),
        in_specs=[_spec_degp, _spec_acc, _spec_rows, _spec_b],
        out_specs=_spec_rows, out_shape=_out_rows,
    )(degp, acc, g, b)


def kernel(x, edge_index, W1, b1, W2, b2):
    src3 = edge_index[0].reshape(NW, NBLK, B)
    dst3 = edge_index[1].reshape(NW, NBLK, B)
    b1r = b1.reshape(1, D)
    b2r = b2.reshape(1, D)

    degp = _deg_parts(dst3).T
    g1 = _tc_pre(degp, x, W1)
    acc1 = _edge_pass(g1, src3, dst3)
    g2 = _tc_mid(degp, acc1, g1, b1r, W2)
    acc2 = _edge_pass(g2, src3, dst3)
    return _tc_post(degp, acc2, g2, b2r)
